# Initial kernel scaffold; baseline (speedup 1.0000x reference)
import jax
import jax.numpy as jnp
from jax.experimental import pallas as pl


def kernel(source, target, num_samples, spacing):
    raise NotImplementedError("write your pallas kernel here")


# speedup vs baseline: 6.5012x; 6.5012x over previous
import functools
import jax
import jax.numpy as jnp
from jax.experimental import pallas as pl
from jax.experimental.pallas import tpu as pltpu

_K = 32


def _topk_body(t_ref, s_ref, vals_ref, idx_ref):
    t = t_ref[0]
    s = s_ref[0]
    r0 = jnp.sum(t * t, axis=1, keepdims=True)
    r1 = jnp.sum(s * s, axis=1, keepdims=True).T
    dots = jax.lax.dot_general(t, s, (((1,), (1,)), ((), ())),
                               preferred_element_type=jnp.float32)
    x = 2.0 * dots - r0 - r1
    tm, n = x.shape
    col = jax.lax.broadcasted_iota(jnp.int32, (tm, n), 1)
    kcol = jax.lax.broadcasted_iota(jnp.int32, (tm, _K), 1)
    vals = jnp.full((tm, _K), -jnp.inf, dtype=jnp.float32)
    idxs = jnp.zeros((tm, _K), dtype=jnp.int32)
    neginf = jnp.float32(-jnp.inf)
    for r in range(_K):
        m = jnp.max(x, axis=1)
        hit = x == m[:, None]
        im = jnp.min(jnp.where(hit, col, n), axis=1)
        x = jnp.where(col == im[:, None], neginf, x)
        sel = kcol == r
        vals = jnp.where(sel, m[:, None], vals)
        idxs = jnp.where(sel, im[:, None], idxs)
    vals_ref[0] = vals
    idx_ref[0] = idxs


def kernel(source, target, num_samples, spacing):
    B, N, D = source.shape
    M = target.shape[1]
    TM = 256
    grid = (B, M // TM)
    vals, idx = pl.pallas_call(
        _topk_body,
        grid=grid,
        in_specs=[
            pl.BlockSpec((1, TM, D), lambda b, i: (b, i, 0)),
            pl.BlockSpec((1, N, D), lambda b, i: (b, 0, 0)),
        ],
        out_specs=[
            pl.BlockSpec((1, TM, _K), lambda b, i: (b, i, 0)),
            pl.BlockSpec((1, TM, _K), lambda b, i: (b, i, 0)),
        ],
        out_shape=[
            jax.ShapeDtypeStruct((B, M, _K), jnp.float32),
            jax.ShapeDtypeStruct((B, M, _K), jnp.int32),
        ],
    )(target, source)
    dep = (jnp.asarray(num_samples, dtype=idx.dtype) - _K) + jnp.asarray(spacing, dtype=idx.dtype)
    p_idx = idx + dep
    batch_idx = jnp.broadcast_to(
        jnp.arange(B, dtype=p_idx.dtype)[:, None, None], (B, M, _K))
    patches_idx = jnp.stack([batch_idx, p_idx], axis=-1)
    return patches_idx, vals


# speedup vs baseline: 19.2055x; 2.9542x over previous
import functools
import jax
import jax.numpy as jnp
from jax import lax
from jax.experimental import pallas as pl
from jax.experimental.pallas import tpu as pltpu
from jax.experimental.pallas import tpu_sc as plsc

_K = 32
_L = 16
_NW = 32
_W = 8
_NEG_INF = float("-inf")


def _dist_body(t_ref, s_ref, x_ref):
    t = t_ref[0]
    s = s_ref[0]
    r0 = jnp.sum(t * t, axis=1, keepdims=True)
    r1 = jnp.sum(s * s, axis=1, keepdims=True).T
    dots = lax.dot_general(t, s, (((1,), (1,)), ((), ())),
                           preferred_element_type=jnp.float32)
    x_ref[0] = 2.0 * dots - r0 - r1


def _neg_dist(source, target):
    B, N, D = source.shape
    M = target.shape[1]
    TM = 256
    return pl.pallas_call(
        _dist_body,
        grid=(B, M // TM),
        in_specs=[
            pl.BlockSpec((1, TM, D), lambda b, i: (b, i, 0)),
            pl.BlockSpec((1, N, D), lambda b, i: (b, 0, 0)),
        ],
        out_specs=pl.BlockSpec((1, TM, N), lambda b, i: (b, i, 0)),
        out_shape=jax.ShapeDtypeStruct((B, M, N), jnp.float32),
    )(target, source)


def _merge16(carry, vk, vv):
    r0k, r0v, r1k, r1v = carry
    vk, vv = plsc.sort_key_val(vk, vv)
    c = r1k >= vk
    hk = jnp.where(c, r1k, vk)
    hv = jnp.where(c, r1v, vv)
    hk, hv = plsc.sort_key_val(hk, hv)
    c2 = r0k >= hk
    pk = jnp.where(c2, r0k, hk)
    pv = jnp.where(c2, r0v, hv)
    qk = jnp.where(c2, hk, r0k)
    qv = jnp.where(c2, hv, r0v)
    r0k, r0v = plsc.sort_key_val(pk, pv, descending=True)
    r1k, r1v = plsc.sort_key_val(qk, qv, descending=True)
    return (r0k, r0v, r1k, r1v)


def _make_sc_topk(BM, N):
    rows_per_w = BM // _NW
    nwin = rows_per_w // _W
    SB = 128
    wins_per_blk = SB // _W
    pairs_per_blk = wins_per_blk // 2
    mesh = plsc.VectorSubcoreMesh(core_axis_name="c", subcore_axis_name="s")

    @functools.partial(
        pl.kernel,
        out_type=[
            jax.ShapeDtypeStruct((BM, _K), jnp.float32),
            jax.ShapeDtypeStruct((BM, _K), jnp.int32),
        ],
        mesh=mesh,
        compiler_params=pltpu.CompilerParams(needs_layout_passes=False),
        scratch_types=[
            pltpu.VMEM((_W, N), jnp.float32),
            pltpu.VMEM((_W, N), jnp.float32),
            pltpu.VMEM((SB, _K), jnp.float32),
            pltpu.VMEM((SB, _K), jnp.int32),
            pltpu.VMEM((N // _L,), jnp.float32),
            pltpu.SemaphoreType.DMA,
            pltpu.SemaphoreType.DMA,
        ],
    )
    def sc_topk(x_hbm, vals_hbm, idx_hbm, buf0, buf1, vstage, istage,
                cmref, sem0, sem1):
        cid = lax.axis_index("c")
        sid = lax.axis_index("s")
        wid = sid * 2 + cid
        row_base = wid * rows_per_w
        iota = lax.iota(jnp.int32, _L)
        ninf = jnp.full((_L,), _NEG_INF, jnp.float32)
        zero = jnp.zeros((_L,), jnp.int32)

        def in_slice(win):
            return x_hbm.at[pl.ds(row_base + win * _W, _W), :]

        def process_row(buf, r, lrow):
            def g_body(g, _):
                base = g * (_L * _L)
                m = buf[r, pl.ds(base, _L)]
                for j in range(1, _L):
                    m = jnp.maximum(m, buf[r, pl.ds(base + j * _L, _L)])
                cmref[pl.ds(g * _L, _L)] = m
                return 0
            lax.fori_loop(0, _L, g_body, 0, unroll=False)

            r0k, r0v = plsc.sort_key_val(cmref[pl.ds(0, _L)], iota,
                                         descending=True)
            carry = (r0k, r0v, ninf, zero)

            def g2_body(g, carry):
                return _merge16(carry, cmref[pl.ds(g * _L, _L)],
                                g * _L + iota)
            carry = lax.fori_loop(1, _L, g2_body, carry, unroll=False)
            cols0 = carry[1]
            cols1 = carry[3]

            base0 = ((cols0 >> 4) << 8) | (cols0 & 15)
            base1 = ((cols1 >> 4) << 8) | (cols1 & 15)
            rsplat = jnp.full((_L,), r, jnp.int32)
            fin = (ninf, zero, ninf, zero)

            def j_body(j, fin):
                a0 = base0 + j * _L
                fin = _merge16(fin, plsc.load_gather(buf, [rsplat, a0]), a0)
                a1 = base1 + j * _L
                fin = _merge16(fin, plsc.load_gather(buf, [rsplat, a1]), a1)
                return fin
            f0k, f0v, f1k, f1v = lax.fori_loop(0, _L, j_body, fin,
                                               unroll=False)

            vstage[lrow, pl.ds(0, _L)] = f0k
            vstage[lrow, pl.ds(_L, _L)] = f1k
            istage[lrow, pl.ds(0, _L)] = f0v
            istage[lrow, pl.ds(_L, _L)] = f1v

        def process(buf, win):
            def r_body(r, _):
                process_row(buf, r, lax.rem(win, wins_per_blk) * _W + r)
                return 0
            lax.fori_loop(0, _W, r_body, 0, unroll=False)

        pltpu.async_copy(in_slice(0), buf0, sem0)

        def pair_body(t, _):
            win0 = 2 * t
            pltpu.async_copy(in_slice(win0 + 1), buf1, sem1)
            pltpu.make_async_copy(in_slice(win0), buf0, sem0).wait()
            process(buf0, win0)

            @pl.when(win0 + 2 < nwin)
            def _():
                pltpu.async_copy(in_slice(win0 + 2), buf0, sem0)

            pltpu.make_async_copy(in_slice(win0 + 1), buf1, sem1).wait()
            process(buf1, win0 + 1)

            @pl.when(lax.rem(t, pairs_per_blk) == pairs_per_blk - 1)
            def _():
                out0 = row_base + (t // pairs_per_blk) * SB
                pltpu.sync_copy(vstage, vals_hbm.at[pl.ds(out0, SB), :])
                pltpu.sync_copy(istage, idx_hbm.at[pl.ds(out0, SB), :])
            return 0
        lax.fori_loop(0, nwin // 2, pair_body, 0, unroll=False)

    return sc_topk


def kernel(source, target, num_samples, spacing):
    B, N, D = source.shape
    M = target.shape[1]
    x = _neg_dist(source, target)
    xf = x.reshape(B * M, N)
    vals, idx = _make_sc_topk(B * M, N)(xf)
    vals = vals.reshape(B, M, _K)
    idx = idx.reshape(B, M, _K)
    dep = (jnp.asarray(num_samples, dtype=idx.dtype) - _K) + jnp.asarray(
        spacing, dtype=idx.dtype)
    p_idx = idx + dep
    batch_idx = jnp.broadcast_to(
        jnp.arange(B, dtype=p_idx.dtype)[:, None, None], (B, M, _K))
    patches_idx = jnp.stack([batch_idx, p_idx], axis=-1)
    return patches_idx, vals


# speedup vs baseline: 35.4572x; 1.8462x over previous
import functools
import jax
import jax.numpy as jnp
from jax import lax
from jax.experimental import pallas as pl
from jax.experimental.pallas import tpu as pltpu
from jax.experimental.pallas import tpu_sc as plsc

_K = 32
_L = 16
_NW = 32
_W = 8
_NEG_INF = float("-inf")


def _dist_body(t_ref, s_ref, x_ref):
    t = t_ref[0]
    s = s_ref[0]
    r0 = jnp.sum(t * t, axis=1, keepdims=True)
    r1 = jnp.sum(s * s, axis=1, keepdims=True).T
    dots = lax.dot_general(t, s, (((1,), (1,)), ((), ())),
                           preferred_element_type=jnp.float32)
    x_ref[0] = 2.0 * dots - r0 - r1


def _neg_dist(source, target):
    B, N, D = source.shape
    M = target.shape[1]
    TM = 256
    return pl.pallas_call(
        _dist_body,
        grid=(B, M // TM),
        in_specs=[
            pl.BlockSpec((1, TM, D), lambda b, i: (b, i, 0)),
            pl.BlockSpec((1, N, D), lambda b, i: (b, 0, 0)),
        ],
        out_specs=pl.BlockSpec((1, TM, N), lambda b, i: (b, i, 0)),
        out_shape=jax.ShapeDtypeStruct((B, M, N), jnp.float32),
    )(target, source)


def _merge16(carry, vk, vv):
    r0k, r0v, r1k, r1v = carry
    vk, vv = plsc.sort_key_val(vk, vv)
    c = r1k >= vk
    hk = jnp.where(c, r1k, vk)
    hv = jnp.where(c, r1v, vv)
    hk, hv = plsc.sort_key_val(hk, hv)
    c2 = r0k >= hk
    pk = jnp.where(c2, r0k, hk)
    pv = jnp.where(c2, r0v, hv)
    qk = jnp.where(c2, hk, r0k)
    qv = jnp.where(c2, hv, r0v)
    r0k, r0v = plsc.sort_key_val(pk, pv, descending=True)
    r1k, r1v = plsc.sort_key_val(qk, qv, descending=True)
    return (r0k, r0v, r1k, r1v)


def _make_sc_topk(BM, N):
    rows_per_w = BM // _NW
    nwin = rows_per_w // _W
    SB = 128
    wins_per_blk = SB // _W
    pairs_per_blk = wins_per_blk // 2
    mesh = plsc.VectorSubcoreMesh(core_axis_name="c", subcore_axis_name="s")

    @functools.partial(
        pl.kernel,
        out_type=[
            jax.ShapeDtypeStruct((BM, _K), jnp.float32),
            jax.ShapeDtypeStruct((BM, _K), jnp.int32),
        ],
        mesh=mesh,
        compiler_params=pltpu.CompilerParams(needs_layout_passes=False),
        scratch_types=[
            pltpu.VMEM((_W, N), jnp.float32),
            pltpu.VMEM((_W, N), jnp.float32),
            pltpu.VMEM((SB, _K), jnp.float32),
            pltpu.VMEM((SB, _K), jnp.int32),
            pltpu.VMEM((4 * (N // _L),), jnp.float32),
            pltpu.SemaphoreType.DMA,
            pltpu.SemaphoreType.DMA,
        ],
    )
    def sc_topk(x_hbm, vals_hbm, idx_hbm, buf0, buf1, vstage, istage,
                cmref, sem0, sem1):
        cid = lax.axis_index("c")
        sid = lax.axis_index("s")
        wid = sid * 2 + cid
        row_base = wid * rows_per_w
        iota = lax.iota(jnp.int32, _L)
        ninf = jnp.full((_L,), _NEG_INF, jnp.float32)
        zero = jnp.zeros((_L,), jnp.int32)

        def in_slice(win):
            return x_hbm.at[pl.ds(row_base + win * _W, _W), :]

        NCOL = N // _L
        NR = 4

        def process_quad(buf, r0, lrow0):
            def g_body(g, _):
                base = g * (_L * _L)
                for k in range(NR):
                    m = buf[r0 + k, pl.ds(base, _L)]
                    for j in range(1, _L):
                        m = jnp.maximum(m, buf[r0 + k,
                                               pl.ds(base + j * _L, _L)])
                    cmref[pl.ds(k * NCOL + g * _L, _L)] = m
                return 0
            lax.fori_loop(0, _L, g_body, 0, unroll=False)

            carry = []
            for k in range(NR):
                r0k, r0v = plsc.sort_key_val(cmref[pl.ds(k * NCOL, _L)],
                                             iota, descending=True)
                carry += [r0k, r0v, ninf, zero]

            def g2_body(g, carry):
                ids = g * _L + iota
                out = []
                for k in range(NR):
                    out += list(_merge16(
                        tuple(carry[4 * k:4 * k + 4]),
                        cmref[pl.ds(k * NCOL + g * _L, _L)], ids))
                return tuple(out)
            carry = lax.fori_loop(1, _L, g2_body, tuple(carry),
                                  unroll=False)

            bases = []
            rsplats = []
            fin = []
            for k in range(NR):
                cols0 = carry[4 * k + 1]
                cols1 = carry[4 * k + 3]
                bases.append((((cols0 >> 4) << 8) | (cols0 & 15),
                              ((cols1 >> 4) << 8) | (cols1 & 15)))
                rsplats.append(jnp.full((_L,), r0 + k, jnp.int32))
                fin += [ninf, zero, ninf, zero]

            def j_body(j, fin):
                out = []
                for k in range(NR):
                    fk = tuple(fin[4 * k:4 * k + 4])
                    a0 = bases[k][0] + j * _L
                    fk = _merge16(fk, plsc.load_gather(buf, [rsplats[k], a0]),
                                  a0)
                    a1 = bases[k][1] + j * _L
                    fk = _merge16(fk, plsc.load_gather(buf, [rsplats[k], a1]),
                                  a1)
                    out += list(fk)
                return tuple(out)
            fin = lax.fori_loop(0, _L, j_body, tuple(fin), unroll=False)

            for k in range(NR):
                vstage[lrow0 + k, pl.ds(0, _L)] = fin[4 * k]
                vstage[lrow0 + k, pl.ds(_L, _L)] = fin[4 * k + 2]
                istage[lrow0 + k, pl.ds(0, _L)] = fin[4 * k + 1]
                istage[lrow0 + k, pl.ds(_L, _L)] = fin[4 * k + 3]

        def process(buf, win):
            def q_body(q, _):
                r0 = q * NR
                process_quad(buf, r0, lax.rem(win, wins_per_blk) * _W + r0)
                return 0
            lax.fori_loop(0, _W // NR, q_body, 0, unroll=False)

        pltpu.async_copy(in_slice(0), buf0, sem0)

        def pair_body(t, _):
            win0 = 2 * t
            pltpu.async_copy(in_slice(win0 + 1), buf1, sem1)
            pltpu.make_async_copy(in_slice(win0), buf0, sem0).wait()
            process(buf0, win0)

            @pl.when(win0 + 2 < nwin)
            def _():
                pltpu.async_copy(in_slice(win0 + 2), buf0, sem0)

            pltpu.make_async_copy(in_slice(win0 + 1), buf1, sem1).wait()
            process(buf1, win0 + 1)

            @pl.when(lax.rem(t, pairs_per_blk) == pairs_per_blk - 1)
            def _():
                out0 = row_base + (t // pairs_per_blk) * SB
                pltpu.sync_copy(vstage, vals_hbm.at[pl.ds(out0, SB), :])
                pltpu.sync_copy(istage, idx_hbm.at[pl.ds(out0, SB), :])
            return 0
        lax.fori_loop(0, nwin // 2, pair_body, 0, unroll=False)

    return sc_topk


def kernel(source, target, num_samples, spacing):
    B, N, D = source.shape
    M = target.shape[1]
    x = _neg_dist(source, target)
    xf = x.reshape(B * M, N)
    vals, idx = _make_sc_topk(B * M, N)(xf)
    vals = vals.reshape(B, M, _K)
    idx = idx.reshape(B, M, _K)
    dep = (jnp.asarray(num_samples, dtype=idx.dtype) - _K) + jnp.asarray(
        spacing, dtype=idx.dtype)
    p_idx = idx + dep
    batch_idx = jnp.broadcast_to(
        jnp.arange(B, dtype=p_idx.dtype)[:, None, None], (B, M, _K))
    patches_idx = jnp.stack([batch_idx, p_idx], axis=-1)
    return patches_idx, vals


# speedup vs baseline: 38.9010x; 1.0971x over previous
import functools
import jax
import jax.numpy as jnp
from jax import lax
from jax.experimental import pallas as pl
from jax.experimental.pallas import tpu as pltpu
from jax.experimental.pallas import tpu_sc as plsc

_K = 32
_L = 16
_NW = 32
_W = 8
_NEG_INF = float("-inf")


def _dist_body(t_ref, s_ref, x_ref, cm_ref):
    t = t_ref[0]
    s = s_ref[0]
    r0 = jnp.sum(t * t, axis=1, keepdims=True)
    r1 = jnp.sum(s * s, axis=1, keepdims=True).T
    dots = lax.dot_general(t, s, (((1,), (1,)), ((), ())),
                           preferred_element_type=jnp.float32)
    x = 2.0 * dots - r0 - r1
    x_ref[0] = x
    tm, n = x.shape
    cm_ref[0] = jnp.max(x.reshape(tm, 2, _L, 128), axis=2).reshape(tm, 256)


def _neg_dist(source, target):
    B, N, D = source.shape
    M = target.shape[1]
    TM = 256
    return pl.pallas_call(
        _dist_body,
        grid=(B, M // TM),
        in_specs=[
            pl.BlockSpec((1, TM, D), lambda b, i: (b, i, 0)),
            pl.BlockSpec((1, N, D), lambda b, i: (b, 0, 0)),
        ],
        out_specs=[
            pl.BlockSpec((1, TM, N), lambda b, i: (b, i, 0)),
            pl.BlockSpec((1, TM, N // _L), lambda b, i: (b, i, 0)),
        ],
        out_shape=[
            jax.ShapeDtypeStruct((B, M, N), jnp.float32),
            jax.ShapeDtypeStruct((B, M, N // _L), jnp.float32),
        ],
    )(target, source)


def _merge16(carry, vk, vv):
    r0k, r0v, r1k, r1v = carry
    vk, vv = plsc.sort_key_val(vk, vv)
    c = r1k >= vk
    hk = jnp.where(c, r1k, vk)
    hv = jnp.where(c, r1v, vv)
    hk, hv = plsc.sort_key_val(hk, hv)
    c2 = r0k >= hk
    pk = jnp.where(c2, r0k, hk)
    pv = jnp.where(c2, r0v, hv)
    qk = jnp.where(c2, hk, r0k)
    qv = jnp.where(c2, hv, r0v)
    r0k, r0v = plsc.sort_key_val(pk, pv, descending=True)
    r1k, r1v = plsc.sort_key_val(qk, qv, descending=True)
    return (r0k, r0v, r1k, r1v)


def _make_sc_topk(BM, N):
    rows_per_w = BM // _NW
    nwin = rows_per_w // _W
    SB = 128
    wins_per_blk = SB // _W
    pairs_per_blk = wins_per_blk // 2
    mesh = plsc.VectorSubcoreMesh(core_axis_name="c", subcore_axis_name="s")

    @functools.partial(
        pl.kernel,
        out_type=[
            jax.ShapeDtypeStruct((BM, _K), jnp.float32),
            jax.ShapeDtypeStruct((BM, _K), jnp.int32),
        ],
        mesh=mesh,
        compiler_params=pltpu.CompilerParams(needs_layout_passes=False),
        scratch_types=[
            pltpu.VMEM((_W, N), jnp.float32),
            pltpu.VMEM((_W, N), jnp.float32),
            pltpu.VMEM((_W, N // _L), jnp.float32),
            pltpu.VMEM((_W, N // _L), jnp.float32),
            pltpu.VMEM((SB, _K), jnp.float32),
            pltpu.VMEM((SB, _K), jnp.int32),
            pltpu.SemaphoreType.DMA,
            pltpu.SemaphoreType.DMA,
            pltpu.SemaphoreType.DMA,
            pltpu.SemaphoreType.DMA,
        ],
    )
    def sc_topk(x_hbm, cm_hbm, vals_hbm, idx_hbm, buf0, buf1, cmbuf0,
                cmbuf1, vstage, istage, sem0, sem1, csem0, csem1):
        cid = lax.axis_index("c")
        sid = lax.axis_index("s")
        wid = sid * 2 + cid
        row_base = wid * rows_per_w
        iota = lax.iota(jnp.int32, _L)
        ninf = jnp.full((_L,), _NEG_INF, jnp.float32)
        zero = jnp.zeros((_L,), jnp.int32)

        def in_slice(win):
            return x_hbm.at[pl.ds(row_base + win * _W, _W), :]

        def cm_slice(win):
            return cm_hbm.at[pl.ds(row_base + win * _W, _W), :]

        NR = 4

        def process_quad(buf, cmbuf, r0, lrow0):
            carry = []
            for k in range(NR):
                r0k, r0v = plsc.sort_key_val(cmbuf[r0 + k, pl.ds(0, _L)],
                                             iota, descending=True)
                carry += [r0k, r0v, ninf, zero]

            def g2_body(g, carry):
                ids = g * _L + iota
                out = []
                for k in range(NR):
                    out += list(_merge16(
                        tuple(carry[4 * k:4 * k + 4]),
                        cmbuf[r0 + k, pl.ds(g * _L, _L)], ids))
                return tuple(out)
            carry = lax.fori_loop(1, _L, g2_body, tuple(carry),
                                  unroll=False)

            bases = []
            rsplats = []
            fin = []
            for k in range(NR):
                cols0 = carry[4 * k + 1]
                cols1 = carry[4 * k + 3]
                bases.append((((cols0 >> 7) << 11) | (cols0 & 127),
                              ((cols1 >> 7) << 11) | (cols1 & 127)))
                rsplats.append(jnp.full((_L,), r0 + k, jnp.int32))
                fin += [ninf, zero, ninf, zero]

            def j_body(j, fin):
                out = []
                for k in range(NR):
                    fk = tuple(fin[4 * k:4 * k + 4])
                    a0 = bases[k][0] + j * 128
                    fk = _merge16(fk, plsc.load_gather(buf, [rsplats[k], a0]),
                                  a0)
                    a1 = bases[k][1] + j * 128
                    fk = _merge16(fk, plsc.load_gather(buf, [rsplats[k], a1]),
                                  a1)
                    out += list(fk)
                return tuple(out)
            fin = lax.fori_loop(0, _L, j_body, tuple(fin), unroll=False)

            for k in range(NR):
                vstage[lrow0 + k, pl.ds(0, _L)] = fin[4 * k]
                vstage[lrow0 + k, pl.ds(_L, _L)] = fin[4 * k + 2]
                istage[lrow0 + k, pl.ds(0, _L)] = fin[4 * k + 1]
                istage[lrow0 + k, pl.ds(_L, _L)] = fin[4 * k + 3]

        def process(buf, cmbuf, win):
            def q_body(q, _):
                r0 = q * NR
                process_quad(buf, cmbuf, r0,
                             lax.rem(win, wins_per_blk) * _W + r0)
                return 0
            lax.fori_loop(0, _W // NR, q_body, 0, unroll=False)

        pltpu.async_copy(in_slice(0), buf0, sem0)
        pltpu.async_copy(cm_slice(0), cmbuf0, csem0)

        def pair_body(t, _):
            win0 = 2 * t
            pltpu.async_copy(in_slice(win0 + 1), buf1, sem1)
            pltpu.async_copy(cm_slice(win0 + 1), cmbuf1, csem1)
            pltpu.make_async_copy(in_slice(win0), buf0, sem0).wait()
            pltpu.make_async_copy(cm_slice(win0), cmbuf0, csem0).wait()
            process(buf0, cmbuf0, win0)

            @pl.when(win0 + 2 < nwin)
            def _():
                pltpu.async_copy(in_slice(win0 + 2), buf0, sem0)
                pltpu.async_copy(cm_slice(win0 + 2), cmbuf0, csem0)

            pltpu.make_async_copy(in_slice(win0 + 1), buf1, sem1).wait()
            pltpu.make_async_copy(cm_slice(win0 + 1), cmbuf1, csem1).wait()
            process(buf1, cmbuf1, win0 + 1)

            @pl.when(lax.rem(t, pairs_per_blk) == pairs_per_blk - 1)
            def _():
                out0 = row_base + (t // pairs_per_blk) * SB
                pltpu.sync_copy(vstage, vals_hbm.at[pl.ds(out0, SB), :])
                pltpu.sync_copy(istage, idx_hbm.at[pl.ds(out0, SB), :])
            return 0
        lax.fori_loop(0, nwin // 2, pair_body, 0, unroll=False)

    return sc_topk


def kernel(source, target, num_samples, spacing):
    B, N, D = source.shape
    M = target.shape[1]
    x, cm = _neg_dist(source, target)
    xf = x.reshape(B * M, N)
    cmf = cm.reshape(B * M, N // _L)
    vals, idx = _make_sc_topk(B * M, N)(xf, cmf)
    vals = vals.reshape(B, M, _K)
    idx = idx.reshape(B, M, _K)
    dep = (jnp.asarray(num_samples, dtype=idx.dtype) - _K) + jnp.asarray(
        spacing, dtype=idx.dtype)
    p_idx = idx + dep
    batch_idx = jnp.broadcast_to(
        jnp.arange(B, dtype=p_idx.dtype)[:, None, None], (B, M, _K))
    patches_idx = jnp.stack([batch_idx, p_idx], axis=-1)
    return patches_idx, vals


# speedup vs baseline: 45.1560x; 1.1608x over previous
import functools
import jax
import jax.numpy as jnp
from jax import lax
from jax.experimental import pallas as pl
from jax.experimental.pallas import tpu as pltpu
from jax.experimental.pallas import tpu_sc as plsc

_K = 32
_L = 16
_NW = 32
_W = 8
_NEG_INF = float("-inf")


def _dist_body(t_ref, s_ref, x_ref, cm_ref):
    t = t_ref[0]
    s = s_ref[0]
    r0 = jnp.sum(t * t, axis=1, keepdims=True)
    r1 = jnp.sum(s * s, axis=1, keepdims=True).T
    dots = lax.dot_general(t, s, (((1,), (1,)), ((), ())),
                           preferred_element_type=jnp.float32)
    x = 2.0 * dots - r0 - r1
    x_ref[0] = x
    for h in range(2):
        m = x[:, h * 2048:h * 2048 + 128]
        for v in range(1, _L):
            off = h * 2048 + v * 128
            m = jnp.maximum(m, x[:, off:off + 128])
        cm_ref[0, :, h * 128:(h + 1) * 128] = m


def _neg_dist(source, target):
    B, N, D = source.shape
    M = target.shape[1]
    TM = 256
    return pl.pallas_call(
        _dist_body,
        grid=(B, M // TM),
        in_specs=[
            pl.BlockSpec((1, TM, D), lambda b, i: (b, i, 0)),
            pl.BlockSpec((1, N, D), lambda b, i: (b, 0, 0)),
        ],
        out_specs=[
            pl.BlockSpec((1, TM, N), lambda b, i: (b, i, 0)),
            pl.BlockSpec((1, TM, N // _L), lambda b, i: (b, i, 0)),
        ],
        out_shape=[
            jax.ShapeDtypeStruct((B, M, N), jnp.float32),
            jax.ShapeDtypeStruct((B, M, N // _L), jnp.float32),
        ],
    )(target, source)


def _merge16(carry, vk, vv):
    r0k, r0v, r1k, r1v = carry
    vk, vv = plsc.sort_key_val(vk, vv)
    c = r1k >= vk
    hk = jnp.where(c, r1k, vk)
    hv = jnp.where(c, r1v, vv)
    hk, hv = plsc.sort_key_val(hk, hv)
    c2 = r0k >= hk
    pk = jnp.where(c2, r0k, hk)
    pv = jnp.where(c2, r0v, hv)
    qk = jnp.where(c2, hk, r0k)
    qv = jnp.where(c2, hv, r0v)
    r0k, r0v = plsc.sort_key_val(pk, pv, descending=True)
    r1k, r1v = plsc.sort_key_val(qk, qv, descending=True)
    return (r0k, r0v, r1k, r1v)


def _make_sc_topk(BM, N):
    rows_per_w = BM // _NW
    nwin = rows_per_w // _W
    SB = 128
    wins_per_blk = SB // _W
    pairs_per_blk = wins_per_blk // 2
    mesh = plsc.VectorSubcoreMesh(core_axis_name="c", subcore_axis_name="s")

    @functools.partial(
        pl.kernel,
        out_type=[
            jax.ShapeDtypeStruct((BM, _K), jnp.float32),
            jax.ShapeDtypeStruct((BM, _K), jnp.int32),
        ],
        mesh=mesh,
        compiler_params=pltpu.CompilerParams(needs_layout_passes=False),
        scratch_types=[
            pltpu.VMEM((_W, N), jnp.float32),
            pltpu.VMEM((_W, N), jnp.float32),
            pltpu.VMEM((_W, N // _L), jnp.float32),
            pltpu.VMEM((_W, N // _L), jnp.float32),
            pltpu.VMEM((SB, _K), jnp.float32),
            pltpu.VMEM((SB, _K), jnp.int32),
            pltpu.SemaphoreType.DMA,
            pltpu.SemaphoreType.DMA,
            pltpu.SemaphoreType.DMA,
            pltpu.SemaphoreType.DMA,
        ],
    )
    def sc_topk(x_hbm, cm_hbm, vals_hbm, idx_hbm, buf0, buf1, cmbuf0,
                cmbuf1, vstage, istage, sem0, sem1, csem0, csem1):
        cid = lax.axis_index("c")
        sid = lax.axis_index("s")
        wid = sid * 2 + cid
        row_base = wid * rows_per_w
        iota = lax.iota(jnp.int32, _L)
        ninf = jnp.full((_L,), _NEG_INF, jnp.float32)
        zero = jnp.zeros((_L,), jnp.int32)

        def in_slice(win):
            return x_hbm.at[pl.ds(row_base + win * _W, _W), :]

        def cm_slice(win):
            return cm_hbm.at[pl.ds(row_base + win * _W, _W), :]

        NR = 4

        def process_quad(buf, cmbuf, r0, lrow0):
            carry = []
            for k in range(NR):
                r0k, r0v = plsc.sort_key_val(cmbuf[r0 + k, pl.ds(0, _L)],
                                             iota, descending=True)
                carry += [r0k, r0v, ninf, zero]

            def g2_body(g, carry):
                ids = g * _L + iota
                out = []
                for k in range(NR):
                    out += list(_merge16(
                        tuple(carry[4 * k:4 * k + 4]),
                        cmbuf[r0 + k, pl.ds(g * _L, _L)], ids))
                return tuple(out)
            carry = lax.fori_loop(1, _L, g2_body, tuple(carry),
                                  unroll=False)

            bases = []
            rsplats = []
            fin = []
            for k in range(NR):
                cols0 = carry[4 * k + 1]
                cols1 = carry[4 * k + 3]
                bases.append((((cols0 >> 7) << 11) | (cols0 & 127),
                              ((cols1 >> 7) << 11) | (cols1 & 127)))
                rsplats.append(jnp.full((_L,), r0 + k, jnp.int32))
                fin += [ninf, zero, ninf, zero]

            def j_body(j, fin):
                out = []
                for k in range(NR):
                    fk = tuple(fin[4 * k:4 * k + 4])
                    a0 = bases[k][0] + j * 128
                    fk = _merge16(fk, plsc.load_gather(buf, [rsplats[k], a0]),
                                  a0)
                    a1 = bases[k][1] + j * 128
                    fk = _merge16(fk, plsc.load_gather(buf, [rsplats[k], a1]),
                                  a1)
                    out += list(fk)
                return tuple(out)
            fin = lax.fori_loop(0, _L, j_body, tuple(fin), unroll=False)

            for k in range(NR):
                vstage[lrow0 + k, pl.ds(0, _L)] = fin[4 * k]
                vstage[lrow0 + k, pl.ds(_L, _L)] = fin[4 * k + 2]
                istage[lrow0 + k, pl.ds(0, _L)] = fin[4 * k + 1]
                istage[lrow0 + k, pl.ds(_L, _L)] = fin[4 * k + 3]

        def process(buf, cmbuf, win):
            def q_body(q, _):
                r0 = q * NR
                process_quad(buf, cmbuf, r0,
                             lax.rem(win, wins_per_blk) * _W + r0)
                return 0
            lax.fori_loop(0, _W // NR, q_body, 0, unroll=False)

        pltpu.async_copy(in_slice(0), buf0, sem0)
        pltpu.async_copy(cm_slice(0), cmbuf0, csem0)

        def pair_body(t, _):
            win0 = 2 * t
            pltpu.async_copy(in_slice(win0 + 1), buf1, sem1)
            pltpu.async_copy(cm_slice(win0 + 1), cmbuf1, csem1)
            pltpu.make_async_copy(in_slice(win0), buf0, sem0).wait()
            pltpu.make_async_copy(cm_slice(win0), cmbuf0, csem0).wait()
            process(buf0, cmbuf0, win0)

            @pl.when(win0 + 2 < nwin)
            def _():
                pltpu.async_copy(in_slice(win0 + 2), buf0, sem0)
                pltpu.async_copy(cm_slice(win0 + 2), cmbuf0, csem0)

            pltpu.make_async_copy(in_slice(win0 + 1), buf1, sem1).wait()
            pltpu.make_async_copy(cm_slice(win0 + 1), cmbuf1, csem1).wait()
            process(buf1, cmbuf1, win0 + 1)

            @pl.when(lax.rem(t, pairs_per_blk) == pairs_per_blk - 1)
            def _():
                out0 = row_base + (t // pairs_per_blk) * SB
                pltpu.sync_copy(vstage, vals_hbm.at[pl.ds(out0, SB), :])
                pltpu.sync_copy(istage, idx_hbm.at[pl.ds(out0, SB), :])
            return 0
        lax.fori_loop(0, nwin // 2, pair_body, 0, unroll=False)

    return sc_topk


def kernel(source, target, num_samples, spacing):
    B, N, D = source.shape
    M = target.shape[1]
    x, cm = _neg_dist(source, target)
    xf = x.reshape(B * M, N)
    cmf = cm.reshape(B * M, N // _L)
    vals, idx = _make_sc_topk(B * M, N)(xf, cmf)
    vals = vals.reshape(B, M, _K)
    idx = idx.reshape(B, M, _K)
    dep = (jnp.asarray(num_samples, dtype=idx.dtype) - _K) + jnp.asarray(
        spacing, dtype=idx.dtype)
    p_idx = idx + dep
    batch_idx = jnp.broadcast_to(
        jnp.arange(B, dtype=p_idx.dtype)[:, None, None], (B, M, _K))
    patches_idx = jnp.stack([batch_idx, p_idx], axis=-1)
    return patches_idx, vals


# speedup vs baseline: 56.3209x; 1.2473x over previous
import functools
import jax
import jax.numpy as jnp
from jax import lax
from jax.experimental import pallas as pl
from jax.experimental.pallas import tpu as pltpu
from jax.experimental.pallas import tpu_sc as plsc

_K = 32
_L = 16
_NW = 32
_W = 8
_NEG_INF = float("-inf")


def _dist_body(t_ref, s_ref, x_ref, cm_ref):
    t = t_ref[0]
    s = s_ref[0]
    r0 = jnp.sum(t * t, axis=1, keepdims=True)
    r1 = jnp.sum(s * s, axis=1, keepdims=True).T
    dots = lax.dot_general(t, s, (((1,), (1,)), ((), ())),
                           preferred_element_type=jnp.float32)
    x = 2.0 * dots - r0 - r1
    x_ref[0] = x
    for h in range(2):
        m = x[:, h * 2048:h * 2048 + 128]
        for v in range(1, _L):
            off = h * 2048 + v * 128
            m = jnp.maximum(m, x[:, off:off + 128])
        cm_ref[0, :, h * 128:(h + 1) * 128] = m


def _neg_dist(source, target, b0, Bc):
    B, N, D = source.shape
    M = target.shape[1]
    TM = 256
    return pl.pallas_call(
        _dist_body,
        grid=(Bc, M // TM),
        in_specs=[
            pl.BlockSpec((1, TM, D), lambda b, i: (b + b0, i, 0)),
            pl.BlockSpec((1, N, D), lambda b, i: (b + b0, 0, 0)),
        ],
        out_specs=[
            pl.BlockSpec((1, TM, N), lambda b, i: (b, i, 0)),
            pl.BlockSpec((1, TM, N // _L), lambda b, i: (b, i, 0)),
        ],
        out_shape=[
            jax.ShapeDtypeStruct((Bc, M, N), jnp.float32),
            jax.ShapeDtypeStruct((Bc, M, N // _L), jnp.float32),
        ],
    )(target, source)


def _merge16(carry, vk, vv):
    r0k, r0v, r1k, r1v = carry
    vk, vv = plsc.sort_key_val(vk, vv)
    c = r1k >= vk
    hk = jnp.where(c, r1k, vk)
    hv = jnp.where(c, r1v, vv)
    hk, hv = plsc.sort_key_val(hk, hv)
    c2 = r0k >= hk
    pk = jnp.where(c2, r0k, hk)
    pv = jnp.where(c2, r0v, hv)
    qk = jnp.where(c2, hk, r0k)
    qv = jnp.where(c2, hv, r0v)
    r0k, r0v = plsc.sort_key_val(pk, pv, descending=True)
    r1k, r1v = plsc.sort_key_val(qk, qv, descending=True)
    return (r0k, r0v, r1k, r1v)


def _make_sc_topk(BM, N):
    rows_per_w = BM // _NW
    nwin = rows_per_w // _W
    SB = 128
    wins_per_blk = SB // _W
    pairs_per_blk = wins_per_blk // 2
    mesh = plsc.VectorSubcoreMesh(core_axis_name="c", subcore_axis_name="s")

    @functools.partial(
        pl.kernel,
        out_type=[
            jax.ShapeDtypeStruct((BM, _K), jnp.float32),
            jax.ShapeDtypeStruct((BM, _K), jnp.int32),
        ],
        mesh=mesh,
        compiler_params=pltpu.CompilerParams(needs_layout_passes=False),
        scratch_types=[
            pltpu.VMEM((_W, N), jnp.float32),
            pltpu.VMEM((_W, N), jnp.float32),
            pltpu.VMEM((_W, N // _L), jnp.float32),
            pltpu.VMEM((_W, N // _L), jnp.float32),
            pltpu.VMEM((SB, _K), jnp.float32),
            pltpu.VMEM((SB, _K), jnp.int32),
            pltpu.SemaphoreType.DMA,
            pltpu.SemaphoreType.DMA,
            pltpu.SemaphoreType.DMA,
            pltpu.SemaphoreType.DMA,
        ],
    )
    def sc_topk(x_hbm, cm_hbm, vals_hbm, idx_hbm, buf0, buf1, cmbuf0,
                cmbuf1, vstage, istage, sem0, sem1, csem0, csem1):
        cid = lax.axis_index("c")
        sid = lax.axis_index("s")
        wid = sid * 2 + cid
        row_base = wid * rows_per_w
        iota = lax.iota(jnp.int32, _L)
        ninf = jnp.full((_L,), _NEG_INF, jnp.float32)
        zero = jnp.zeros((_L,), jnp.int32)

        def in_slice(win):
            return x_hbm.at[pl.ds(row_base + win * _W, _W), :]

        def cm_slice(win):
            return cm_hbm.at[pl.ds(row_base + win * _W, _W), :]

        NR = 4

        def process_quad(buf, cmbuf, r0, lrow0):
            carry = []
            for k in range(NR):
                r0k, r0v = plsc.sort_key_val(cmbuf[r0 + k, pl.ds(0, _L)],
                                             iota, descending=True)
                carry += [r0k, r0v, ninf, zero]

            def g2_body(g, carry):
                ids = g * _L + iota
                out = []
                for k in range(NR):
                    out += list(_merge16(
                        tuple(carry[4 * k:4 * k + 4]),
                        cmbuf[r0 + k, pl.ds(g * _L, _L)], ids))
                return tuple(out)
            carry = lax.fori_loop(1, _L, g2_body, tuple(carry),
                                  unroll=False)

            bases = []
            rsplats = []
            fin = []
            for k in range(NR):
                cols0 = carry[4 * k + 1]
                cols1 = carry[4 * k + 3]
                bases.append((((cols0 >> 7) << 11) | (cols0 & 127),
                              ((cols1 >> 7) << 11) | (cols1 & 127)))
                rsplats.append(jnp.full((_L,), r0 + k, jnp.int32))
                fin += [ninf, zero, ninf, zero]

            def j_body(j, fin):
                out = []
                for k in range(NR):
                    fk = tuple(fin[4 * k:4 * k + 4])
                    a0 = bases[k][0] + j * 128
                    fk = _merge16(fk, plsc.load_gather(buf, [rsplats[k], a0]),
                                  a0)
                    a1 = bases[k][1] + j * 128
                    fk = _merge16(fk, plsc.load_gather(buf, [rsplats[k], a1]),
                                  a1)
                    out += list(fk)
                return tuple(out)
            fin = lax.fori_loop(0, _L, j_body, tuple(fin), unroll=False)

            for k in range(NR):
                vstage[lrow0 + k, pl.ds(0, _L)] = fin[4 * k]
                vstage[lrow0 + k, pl.ds(_L, _L)] = fin[4 * k + 2]
                istage[lrow0 + k, pl.ds(0, _L)] = fin[4 * k + 1]
                istage[lrow0 + k, pl.ds(_L, _L)] = fin[4 * k + 3]

        def process(buf, cmbuf, win):
            def q_body(q, _):
                r0 = q * NR
                process_quad(buf, cmbuf, r0,
                             lax.rem(win, wins_per_blk) * _W + r0)
                return 0
            lax.fori_loop(0, _W // NR, q_body, 0, unroll=False)

        pltpu.async_copy(in_slice(0), buf0, sem0)
        pltpu.async_copy(cm_slice(0), cmbuf0, csem0)

        def pair_body(t, _):
            win0 = 2 * t
            pltpu.async_copy(in_slice(win0 + 1), buf1, sem1)
            pltpu.async_copy(cm_slice(win0 + 1), cmbuf1, csem1)
            pltpu.make_async_copy(in_slice(win0), buf0, sem0).wait()
            pltpu.make_async_copy(cm_slice(win0), cmbuf0, csem0).wait()
            process(buf0, cmbuf0, win0)

            @pl.when(win0 + 2 < nwin)
            def _():
                pltpu.async_copy(in_slice(win0 + 2), buf0, sem0)
                pltpu.async_copy(cm_slice(win0 + 2), cmbuf0, csem0)

            pltpu.make_async_copy(in_slice(win0 + 1), buf1, sem1).wait()
            pltpu.make_async_copy(cm_slice(win0 + 1), cmbuf1, csem1).wait()
            process(buf1, cmbuf1, win0 + 1)

            @pl.when(lax.rem(t, pairs_per_blk) == pairs_per_blk - 1)
            def _():
                out0 = row_base + (t // pairs_per_blk) * SB
                pltpu.sync_copy(vstage, vals_hbm.at[pl.ds(out0, SB), :])
                pltpu.sync_copy(istage, idx_hbm.at[pl.ds(out0, SB), :])
            return 0
        lax.fori_loop(0, nwin // 2, pair_body, 0, unroll=False)

    return sc_topk


def kernel(source, target, num_samples, spacing):
    B, N, D = source.shape
    M = target.shape[1]
    CH = 4
    Bc = B // CH
    sc_topk = _make_sc_topk(Bc * M, N)
    vparts, iparts = [], []
    for c in range(CH):
        x, cm = _neg_dist(source, target, c * Bc, Bc)
        v, i = sc_topk(x.reshape(Bc * M, N), cm.reshape(Bc * M, N // _L))
        vparts.append(v.reshape(Bc, M, _K))
        iparts.append(i.reshape(Bc, M, _K))
    vals = jnp.concatenate(vparts, axis=0)
    idx = jnp.concatenate(iparts, axis=0)
    dep = (jnp.asarray(num_samples, dtype=idx.dtype) - _K) + jnp.asarray(
        spacing, dtype=idx.dtype)
    p_idx = idx + dep
    batch_idx = jnp.broadcast_to(
        jnp.arange(B, dtype=p_idx.dtype)[:, None, None], (B, M, _K))
    patches_idx = jnp.stack([batch_idx, p_idx], axis=-1)
    return patches_idx, vals


# speedup vs baseline: 57.5503x; 1.0218x over previous
import functools
import jax
import jax.numpy as jnp
from jax import lax
from jax.experimental import pallas as pl
from jax.experimental.pallas import tpu as pltpu
from jax.experimental.pallas import tpu_sc as plsc

_K = 32
_L = 16
_NW = 32
_W = 8
_NEG_INF = float("-inf")


def _dist_body(t_ref, s_ref, x_ref, cm_ref):
    t = t_ref[0]
    s = s_ref[0]
    r0 = jnp.sum(t * t, axis=1, keepdims=True)
    r1 = jnp.sum(s * s, axis=1, keepdims=True).T
    dots = lax.dot_general(t, s, (((1,), (1,)), ((), ())),
                           preferred_element_type=jnp.float32)
    x = 2.0 * dots - r0 - r1
    x_ref[0] = x
    for h in range(2):
        m = x[:, h * 2048:h * 2048 + 128]
        for v in range(1, _L):
            off = h * 2048 + v * 128
            m = jnp.maximum(m, x[:, off:off + 128])
        cm_ref[0, :, h * 128:(h + 1) * 128] = m


def _neg_dist(source, target, b0, Bc):
    B, N, D = source.shape
    M = target.shape[1]
    TM = 256
    return pl.pallas_call(
        _dist_body,
        grid=(Bc, M // TM),
        in_specs=[
            pl.BlockSpec((1, TM, D), lambda b, i: (b + b0, i, 0)),
            pl.BlockSpec((1, N, D), lambda b, i: (b + b0, 0, 0)),
        ],
        out_specs=[
            pl.BlockSpec((1, TM, N), lambda b, i: (b, i, 0)),
            pl.BlockSpec((1, TM, N // _L), lambda b, i: (b, i, 0)),
        ],
        out_shape=[
            jax.ShapeDtypeStruct((Bc, M, N), jnp.float32),
            jax.ShapeDtypeStruct((Bc, M, N // _L), jnp.float32),
        ],
    )(target, source)


def _merge16(carry, vk, vv):
    r0k, r0v, r1k, r1v = carry
    vk, vv = plsc.sort_key_val(vk, vv)
    c = r1k >= vk
    hk = jnp.where(c, r1k, vk)
    hv = jnp.where(c, r1v, vv)
    hk, hv = plsc.sort_key_val(hk, hv)
    c2 = r0k >= hk
    pk = jnp.where(c2, r0k, hk)
    pv = jnp.where(c2, r0v, hv)
    qk = jnp.where(c2, hk, r0k)
    qv = jnp.where(c2, hv, r0v)
    r0k, r0v = plsc.sort_key_val(pk, pv, descending=True)
    r1k, r1v = plsc.sort_key_val(qk, qv, descending=True)
    return (r0k, r0v, r1k, r1v)


def _make_sc_topk(BM, N):
    rows_per_w = BM // _NW
    nwin = rows_per_w // _W
    SB = 128
    wins_per_blk = SB // _W
    pairs_per_blk = wins_per_blk // 2
    mesh = plsc.VectorSubcoreMesh(core_axis_name="c", subcore_axis_name="s")

    @functools.partial(
        pl.kernel,
        out_type=[
            jax.ShapeDtypeStruct((BM, _K), jnp.float32),
            jax.ShapeDtypeStruct((BM, _K), jnp.int32),
        ],
        mesh=mesh,
        compiler_params=pltpu.CompilerParams(needs_layout_passes=False),
        scratch_types=[
            pltpu.VMEM((_W, N), jnp.float32),
            pltpu.VMEM((_W, N), jnp.float32),
            pltpu.VMEM((_W, N // _L), jnp.float32),
            pltpu.VMEM((_W, N // _L), jnp.float32),
            pltpu.VMEM((SB, _K), jnp.float32),
            pltpu.VMEM((SB, _K), jnp.int32),
            pltpu.SemaphoreType.DMA,
            pltpu.SemaphoreType.DMA,
            pltpu.SemaphoreType.DMA,
            pltpu.SemaphoreType.DMA,
        ],
    )
    def sc_topk(x_hbm, cm_hbm, vals_hbm, idx_hbm, buf0, buf1, cmbuf0,
                cmbuf1, vstage, istage, sem0, sem1, csem0, csem1):
        cid = lax.axis_index("c")
        sid = lax.axis_index("s")
        wid = sid * 2 + cid
        row_base = wid * rows_per_w
        iota = lax.iota(jnp.int32, _L)
        ninf = jnp.full((_L,), _NEG_INF, jnp.float32)
        zero = jnp.zeros((_L,), jnp.int32)

        def in_slice(win):
            return x_hbm.at[pl.ds(row_base + win * _W, _W), :]

        def cm_slice(win):
            return cm_hbm.at[pl.ds(row_base + win * _W, _W), :]

        NR = 4

        def process_quad(buf, cmbuf, r0, lrow0):
            carry = []
            for k in range(NR):
                r0k, r0v = plsc.sort_key_val(cmbuf[r0 + k, pl.ds(0, _L)],
                                             iota, descending=True)
                carry += [r0k, r0v, ninf, zero]

            def g2_body(g, carry):
                ids = g * _L + iota
                out = []
                for k in range(NR):
                    out += list(_merge16(
                        tuple(carry[4 * k:4 * k + 4]),
                        cmbuf[r0 + k, pl.ds(g * _L, _L)], ids))
                return tuple(out)
            carry = lax.fori_loop(1, _L, g2_body, tuple(carry),
                                  unroll=False)

            bases = []
            rsplats = []
            fin = []
            for k in range(NR):
                cols0 = carry[4 * k + 1]
                cols1 = carry[4 * k + 3]
                bases.append((((cols0 >> 7) << 11) | (cols0 & 127),
                              ((cols1 >> 7) << 11) | (cols1 & 127)))
                rsplats.append(jnp.full((_L,), r0 + k, jnp.int32))
                fin += [ninf, zero, ninf, zero]

            fin = fin + fin

            def j_body(j, fin):
                out = []
                for k in range(NR):
                    fa = tuple(fin[8 * k:8 * k + 4])
                    fb = tuple(fin[8 * k + 4:8 * k + 8])
                    a0 = bases[k][0] + j * 128
                    fa = _merge16(fa, plsc.load_gather(buf, [rsplats[k], a0]),
                                  a0)
                    a1 = bases[k][1] + j * 128
                    fb = _merge16(fb, plsc.load_gather(buf, [rsplats[k], a1]),
                                  a1)
                    out += list(fa) + list(fb)
                return tuple(out)
            fin = lax.fori_loop(0, _L, j_body, tuple(fin), unroll=False)

            for k in range(NR):
                a0k, a0v, a1k, a1v = fin[8 * k:8 * k + 4]
                b0k, b0v, b1k, b1v = fin[8 * k + 4:8 * k + 8]
                rb0k = lax.rev(b1k, (0,))
                rb0v = lax.rev(b1v, (0,))
                rb1k = lax.rev(b0k, (0,))
                rb1v = lax.rev(b0v, (0,))
                c0 = a0k >= rb0k
                p0k = jnp.where(c0, a0k, rb0k)
                p0v = jnp.where(c0, a0v, rb0v)
                c1 = a1k >= rb1k
                p1k = jnp.where(c1, a1k, rb1k)
                p1v = jnp.where(c1, a1v, rb1v)
                c2 = p0k >= p1k
                hik = jnp.where(c2, p0k, p1k)
                hiv = jnp.where(c2, p0v, p1v)
                lok = jnp.where(c2, p1k, p0k)
                lov = jnp.where(c2, p1v, p0v)
                f0k, f0v = plsc.sort_key_val(hik, hiv, descending=True)
                f1k, f1v = plsc.sort_key_val(lok, lov, descending=True)
                vstage[lrow0 + k, pl.ds(0, _L)] = f0k
                vstage[lrow0 + k, pl.ds(_L, _L)] = f1k
                istage[lrow0 + k, pl.ds(0, _L)] = f0v
                istage[lrow0 + k, pl.ds(_L, _L)] = f1v

        def process(buf, cmbuf, win):
            def q_body(q, _):
                r0 = q * NR
                process_quad(buf, cmbuf, r0,
                             lax.rem(win, wins_per_blk) * _W + r0)
                return 0
            lax.fori_loop(0, _W // NR, q_body, 0, unroll=False)

        pltpu.async_copy(in_slice(0), buf0, sem0)
        pltpu.async_copy(cm_slice(0), cmbuf0, csem0)

        def pair_body(t, _):
            win0 = 2 * t
            pltpu.async_copy(in_slice(win0 + 1), buf1, sem1)
            pltpu.async_copy(cm_slice(win0 + 1), cmbuf1, csem1)
            pltpu.make_async_copy(in_slice(win0), buf0, sem0).wait()
            pltpu.make_async_copy(cm_slice(win0), cmbuf0, csem0).wait()
            process(buf0, cmbuf0, win0)

            @pl.when(win0 + 2 < nwin)
            def _():
                pltpu.async_copy(in_slice(win0 + 2), buf0, sem0)
                pltpu.async_copy(cm_slice(win0 + 2), cmbuf0, csem0)

            pltpu.make_async_copy(in_slice(win0 + 1), buf1, sem1).wait()
            pltpu.make_async_copy(cm_slice(win0 + 1), cmbuf1, csem1).wait()
            process(buf1, cmbuf1, win0 + 1)

            @pl.when(lax.rem(t, pairs_per_blk) == pairs_per_blk - 1)
            def _():
                out0 = row_base + (t // pairs_per_blk) * SB
                pltpu.sync_copy(vstage, vals_hbm.at[pl.ds(out0, SB), :])
                pltpu.sync_copy(istage, idx_hbm.at[pl.ds(out0, SB), :])
            return 0
        lax.fori_loop(0, nwin // 2, pair_body, 0, unroll=False)

    return sc_topk


def kernel(source, target, num_samples, spacing):
    B, N, D = source.shape
    M = target.shape[1]
    CH = 4
    Bc = B // CH
    sc_topk = _make_sc_topk(Bc * M, N)
    vparts, iparts = [], []
    for c in range(CH):
        x, cm = _neg_dist(source, target, c * Bc, Bc)
        v, i = sc_topk(x.reshape(Bc * M, N), cm.reshape(Bc * M, N // _L))
        vparts.append(v.reshape(Bc, M, _K))
        iparts.append(i.reshape(Bc, M, _K))
    vals = jnp.concatenate(vparts, axis=0)
    idx = jnp.concatenate(iparts, axis=0)
    dep = (jnp.asarray(num_samples, dtype=idx.dtype) - _K) + jnp.asarray(
        spacing, dtype=idx.dtype)
    p_idx = idx + dep
    batch_idx = jnp.broadcast_to(
        jnp.arange(B, dtype=p_idx.dtype)[:, None, None], (B, M, _K))
    patches_idx = jnp.stack([batch_idx, p_idx], axis=-1)
    return patches_idx, vals


# speedup vs baseline: 57.7643x; 1.0037x over previous
import functools
import jax
import jax.numpy as jnp
from jax import lax
from jax.experimental import pallas as pl
from jax.experimental.pallas import tpu as pltpu
from jax.experimental.pallas import tpu_sc as plsc

_K = 32
_L = 16
_NW = 32
_W = 8
_NEG_INF = float("-inf")


def _dist_body(t_ref, s_ref, x_ref, cm_ref):
    t = t_ref[0]
    s = s_ref[0]
    r0 = jnp.sum(t * t, axis=1, keepdims=True)
    r1 = jnp.sum(s * s, axis=1, keepdims=True).T
    dots = lax.dot_general(t, s, (((1,), (1,)), ((), ())),
                           preferred_element_type=jnp.float32)
    x = 2.0 * dots - r0 - r1
    x_ref[0] = x
    for h in range(2):
        m = x[:, h * 2048:h * 2048 + 128]
        for v in range(1, _L):
            off = h * 2048 + v * 128
            m = jnp.maximum(m, x[:, off:off + 128])
        cm_ref[0, :, h * 128:(h + 1) * 128] = m


def _neg_dist(source, target, b0, Bc):
    B, N, D = source.shape
    M = target.shape[1]
    TM = 256
    return pl.pallas_call(
        _dist_body,
        grid=(Bc, M // TM),
        in_specs=[
            pl.BlockSpec((1, TM, D), lambda b, i: (b + b0, i, 0)),
            pl.BlockSpec((1, N, D), lambda b, i: (b + b0, 0, 0)),
        ],
        out_specs=[
            pl.BlockSpec((1, TM, N), lambda b, i: (b, i, 0)),
            pl.BlockSpec((1, TM, N // _L), lambda b, i: (b, i, 0)),
        ],
        out_shape=[
            jax.ShapeDtypeStruct((Bc, M, N), jnp.float32),
            jax.ShapeDtypeStruct((Bc, M, N // _L), jnp.float32),
        ],
    )(target, source)


def _merge16(carry, vk, vv):
    r0k, r0v, r1k, r1v = carry
    vk, vv = plsc.sort_key_val(vk, vv)
    c = r1k >= vk
    hk = jnp.where(c, r1k, vk)
    hv = jnp.where(c, r1v, vv)
    hk, hv = plsc.sort_key_val(hk, hv)
    c2 = r0k >= hk
    pk = jnp.where(c2, r0k, hk)
    pv = jnp.where(c2, r0v, hv)
    qk = jnp.where(c2, hk, r0k)
    qv = jnp.where(c2, hv, r0v)
    r0k, r0v = plsc.sort_key_val(pk, pv, descending=True)
    r1k, r1v = plsc.sort_key_val(qk, qv, descending=True)
    return (r0k, r0v, r1k, r1v)


def _make_sc_topk(BM, N):
    rows_per_w = BM // _NW
    nwin = rows_per_w // _W
    SB = 128
    wins_per_blk = SB // _W
    pairs_per_blk = wins_per_blk // 2
    mesh = plsc.VectorSubcoreMesh(core_axis_name="c", subcore_axis_name="s")

    @functools.partial(
        pl.kernel,
        out_type=[
            jax.ShapeDtypeStruct((BM, _K), jnp.float32),
            jax.ShapeDtypeStruct((BM, _K), jnp.int32),
        ],
        mesh=mesh,
        compiler_params=pltpu.CompilerParams(needs_layout_passes=False),
        scratch_types=[
            pltpu.VMEM((_W, N), jnp.float32),
            pltpu.VMEM((_W, N), jnp.float32),
            pltpu.VMEM((_W, N // _L), jnp.float32),
            pltpu.VMEM((_W, N // _L), jnp.float32),
            pltpu.VMEM((SB, _K), jnp.float32),
            pltpu.VMEM((SB, _K), jnp.int32),
            pltpu.SemaphoreType.DMA,
            pltpu.SemaphoreType.DMA,
            pltpu.SemaphoreType.DMA,
            pltpu.SemaphoreType.DMA,
        ],
    )
    def sc_topk(x_hbm, cm_hbm, vals_hbm, idx_hbm, buf0, buf1, cmbuf0,
                cmbuf1, vstage, istage, sem0, sem1, csem0, csem1):
        cid = lax.axis_index("c")
        sid = lax.axis_index("s")
        wid = sid * 2 + cid
        row_base = wid * rows_per_w
        iota = lax.iota(jnp.int32, _L)
        ninf = jnp.full((_L,), _NEG_INF, jnp.float32)
        zero = jnp.zeros((_L,), jnp.int32)

        def in_slice(win):
            return x_hbm.at[pl.ds(row_base + win * _W, _W), :]

        def cm_slice(win):
            return cm_hbm.at[pl.ds(row_base + win * _W, _W), :]

        NR = 4

        def process_quad(buf, cmbuf, r0, lrow0):
            carry = []
            for k in range(NR):
                r0k, r0v = plsc.sort_key_val(cmbuf[r0 + k, pl.ds(0, _L)],
                                             iota, descending=True)
                carry += [r0k, r0v, ninf, zero]

            def g2_body(g, carry):
                ids = g * _L + iota
                out = []
                for k in range(NR):
                    out += list(_merge16(
                        tuple(carry[4 * k:4 * k + 4]),
                        cmbuf[r0 + k, pl.ds(g * _L, _L)], ids))
                return tuple(out)
            carry = lax.fori_loop(1, _L, g2_body, tuple(carry),
                                  unroll=False)

            bases = []
            rsplats = []
            fin = []
            for k in range(NR):
                cols0 = carry[4 * k + 1]
                cols1 = carry[4 * k + 3]
                bases.append((((cols0 >> 7) << 11) | (cols0 & 127),
                              ((cols1 >> 7) << 11) | (cols1 & 127)))
                rsplats.append(jnp.full((_L,), r0 + k, jnp.int32))
                fin += [ninf, zero, ninf, zero]

            fin = fin + fin

            def j_body(j, fin):
                out = []
                for k in range(NR):
                    fa = tuple(fin[8 * k:8 * k + 4])
                    fb = tuple(fin[8 * k + 4:8 * k + 8])
                    a0 = bases[k][0] + j * 128
                    fa = _merge16(fa, plsc.load_gather(buf, [rsplats[k], a0]),
                                  a0)
                    a1 = bases[k][1] + j * 128
                    fb = _merge16(fb, plsc.load_gather(buf, [rsplats[k], a1]),
                                  a1)
                    out += list(fa) + list(fb)
                return tuple(out)
            fin = lax.fori_loop(0, _L, j_body, tuple(fin), unroll=False)

            for k in range(NR):
                a0k, a0v, a1k, a1v = fin[8 * k:8 * k + 4]
                b0k, b0v, b1k, b1v = fin[8 * k + 4:8 * k + 8]
                rb0k = lax.rev(b1k, (0,))
                rb0v = lax.rev(b1v, (0,))
                rb1k = lax.rev(b0k, (0,))
                rb1v = lax.rev(b0v, (0,))
                c0 = a0k >= rb0k
                p0k = jnp.where(c0, a0k, rb0k)
                p0v = jnp.where(c0, a0v, rb0v)
                c1 = a1k >= rb1k
                p1k = jnp.where(c1, a1k, rb1k)
                p1v = jnp.where(c1, a1v, rb1v)
                c2 = p0k >= p1k
                hik = jnp.where(c2, p0k, p1k)
                hiv = jnp.where(c2, p0v, p1v)
                lok = jnp.where(c2, p1k, p0k)
                lov = jnp.where(c2, p1v, p0v)
                f0k, f0v = plsc.sort_key_val(hik, hiv, descending=True)
                f1k, f1v = plsc.sort_key_val(lok, lov, descending=True)
                vstage[lrow0 + k, pl.ds(0, _L)] = f0k
                vstage[lrow0 + k, pl.ds(_L, _L)] = f1k
                istage[lrow0 + k, pl.ds(0, _L)] = f0v
                istage[lrow0 + k, pl.ds(_L, _L)] = f1v

        def process(buf, cmbuf, win):
            def q_body(q, _):
                r0 = q * NR
                process_quad(buf, cmbuf, r0,
                             lax.rem(win, wins_per_blk) * _W + r0)
                return 0
            lax.fori_loop(0, _W // NR, q_body, 0, unroll=False)

        pltpu.async_copy(in_slice(0), buf0, sem0)
        pltpu.async_copy(cm_slice(0), cmbuf0, csem0)

        def pair_body(t, _):
            win0 = 2 * t
            pltpu.async_copy(in_slice(win0 + 1), buf1, sem1)
            pltpu.async_copy(cm_slice(win0 + 1), cmbuf1, csem1)
            pltpu.make_async_copy(in_slice(win0), buf0, sem0).wait()
            pltpu.make_async_copy(cm_slice(win0), cmbuf0, csem0).wait()
            process(buf0, cmbuf0, win0)

            @pl.when(win0 + 2 < nwin)
            def _():
                pltpu.async_copy(in_slice(win0 + 2), buf0, sem0)
                pltpu.async_copy(cm_slice(win0 + 2), cmbuf0, csem0)

            pltpu.make_async_copy(in_slice(win0 + 1), buf1, sem1).wait()
            pltpu.make_async_copy(cm_slice(win0 + 1), cmbuf1, csem1).wait()
            process(buf1, cmbuf1, win0 + 1)

            @pl.when(lax.rem(t, pairs_per_blk) == pairs_per_blk - 1)
            def _():
                out0 = row_base + (t // pairs_per_blk) * SB
                pltpu.sync_copy(vstage, vals_hbm.at[pl.ds(out0, SB), :])
                pltpu.sync_copy(istage, idx_hbm.at[pl.ds(out0, SB), :])
            return 0
        lax.fori_loop(0, nwin // 2, pair_body, 0, unroll=False)

    return sc_topk


def kernel(source, target, num_samples, spacing):
    B, N, D = source.shape
    M = target.shape[1]
    CH = 8
    Bc = B // CH
    sc_topk = _make_sc_topk(Bc * M, N)
    vparts, iparts = [], []
    for c in range(CH):
        x, cm = _neg_dist(source, target, c * Bc, Bc)
        v, i = sc_topk(x.reshape(Bc * M, N), cm.reshape(Bc * M, N // _L))
        vparts.append(v.reshape(Bc, M, _K))
        iparts.append(i.reshape(Bc, M, _K))
    vals = jnp.concatenate(vparts, axis=0)
    idx = jnp.concatenate(iparts, axis=0)
    dep = (jnp.asarray(num_samples, dtype=idx.dtype) - _K) + jnp.asarray(
        spacing, dtype=idx.dtype)
    p_idx = idx + dep
    batch_idx = jnp.broadcast_to(
        jnp.arange(B, dtype=p_idx.dtype)[:, None, None], (B, M, _K))
    patches_idx = jnp.stack([batch_idx, p_idx], axis=-1)
    return patches_idx, vals


# speedup vs baseline: 58.2426x; 1.0083x over previous
import functools
import jax
import jax.numpy as jnp
from jax import lax
from jax.experimental import pallas as pl
from jax.experimental.pallas import tpu as pltpu
from jax.experimental.pallas import tpu_sc as plsc

_K = 32
_L = 16
_NW = 32
_W = 8
_NEG_INF = float("-inf")


def _dist_body(t_ref, s_ref, x_ref, cm_ref):
    t = t_ref[0]
    s = s_ref[0]
    r0 = jnp.sum(t * t, axis=1, keepdims=True)
    r1 = jnp.sum(s * s, axis=1, keepdims=True).T
    dots = lax.dot_general(t, s, (((1,), (1,)), ((), ())),
                           preferred_element_type=jnp.float32)
    x = 2.0 * dots - r0 - r1
    x_ref[0] = x
    for h in range(2):
        m = x[:, h * 2048:h * 2048 + 128]
        for v in range(1, _L):
            off = h * 2048 + v * 128
            m = jnp.maximum(m, x[:, off:off + 128])
        cm_ref[0, :, h * 128:(h + 1) * 128] = m


def _neg_dist(source, target, b0, Bc):
    B, N, D = source.shape
    M = target.shape[1]
    TM = 512
    return pl.pallas_call(
        _dist_body,
        grid=(Bc, M // TM),
        in_specs=[
            pl.BlockSpec((1, TM, D), lambda b, i: (b + b0, i, 0)),
            pl.BlockSpec((1, N, D), lambda b, i: (b + b0, 0, 0)),
        ],
        out_specs=[
            pl.BlockSpec((1, TM, N), lambda b, i: (b, i, 0)),
            pl.BlockSpec((1, TM, N // _L), lambda b, i: (b, i, 0)),
        ],
        out_shape=[
            jax.ShapeDtypeStruct((Bc, M, N), jnp.float32),
            jax.ShapeDtypeStruct((Bc, M, N // _L), jnp.float32),
        ],
    )(target, source)


def _merge16(carry, vk, vv):
    r0k, r0v, r1k, r1v = carry
    vk, vv = plsc.sort_key_val(vk, vv)
    c = r1k >= vk
    hk = jnp.where(c, r1k, vk)
    hv = jnp.where(c, r1v, vv)
    hk, hv = plsc.sort_key_val(hk, hv)
    c2 = r0k >= hk
    pk = jnp.where(c2, r0k, hk)
    pv = jnp.where(c2, r0v, hv)
    qk = jnp.where(c2, hk, r0k)
    qv = jnp.where(c2, hv, r0v)
    r0k, r0v = plsc.sort_key_val(pk, pv, descending=True)
    r1k, r1v = plsc.sort_key_val(qk, qv, descending=True)
    return (r0k, r0v, r1k, r1v)


def _make_sc_topk(BM, N):
    rows_per_w = BM // _NW
    nwin = rows_per_w // _W
    SB = 128
    wins_per_blk = SB // _W
    pairs_per_blk = wins_per_blk // 2
    mesh = plsc.VectorSubcoreMesh(core_axis_name="c", subcore_axis_name="s")

    @functools.partial(
        pl.kernel,
        out_type=[
            jax.ShapeDtypeStruct((BM, _K), jnp.float32),
            jax.ShapeDtypeStruct((BM, _K), jnp.int32),
        ],
        mesh=mesh,
        compiler_params=pltpu.CompilerParams(needs_layout_passes=False),
        scratch_types=[
            pltpu.VMEM((_W, N), jnp.float32),
            pltpu.VMEM((_W, N), jnp.float32),
            pltpu.VMEM((_W, N // _L), jnp.float32),
            pltpu.VMEM((_W, N // _L), jnp.float32),
            pltpu.VMEM((SB, _K), jnp.float32),
            pltpu.VMEM((SB, _K), jnp.int32),
            pltpu.SemaphoreType.DMA,
            pltpu.SemaphoreType.DMA,
            pltpu.SemaphoreType.DMA,
            pltpu.SemaphoreType.DMA,
        ],
    )
    def sc_topk(x_hbm, cm_hbm, vals_hbm, idx_hbm, buf0, buf1, cmbuf0,
                cmbuf1, vstage, istage, sem0, sem1, csem0, csem1):
        cid = lax.axis_index("c")
        sid = lax.axis_index("s")
        wid = sid * 2 + cid
        row_base = wid * rows_per_w
        iota = lax.iota(jnp.int32, _L)
        ninf = jnp.full((_L,), _NEG_INF, jnp.float32)
        zero = jnp.zeros((_L,), jnp.int32)

        def in_slice(win):
            return x_hbm.at[pl.ds(row_base + win * _W, _W), :]

        def cm_slice(win):
            return cm_hbm.at[pl.ds(row_base + win * _W, _W), :]

        NR = 4

        def process_quad(buf, cmbuf, r0, lrow0):
            carry = []
            for k in range(NR):
                r0k, r0v = plsc.sort_key_val(cmbuf[r0 + k, pl.ds(0, _L)],
                                             iota, descending=True)
                carry += [r0k, r0v, ninf, zero]

            def g2_body(g, carry):
                ids = g * _L + iota
                out = []
                for k in range(NR):
                    out += list(_merge16(
                        tuple(carry[4 * k:4 * k + 4]),
                        cmbuf[r0 + k, pl.ds(g * _L, _L)], ids))
                return tuple(out)
            carry = lax.fori_loop(1, _L, g2_body, tuple(carry),
                                  unroll=False)

            bases = []
            rsplats = []
            fin = []
            for k in range(NR):
                cols0 = carry[4 * k + 1]
                cols1 = carry[4 * k + 3]
                bases.append((((cols0 >> 7) << 11) | (cols0 & 127),
                              ((cols1 >> 7) << 11) | (cols1 & 127)))
                rsplats.append(jnp.full((_L,), r0 + k, jnp.int32))
                fin += [ninf, zero, ninf, zero]

            fin = fin + fin

            def j_body(j, fin):
                out = []
                for k in range(NR):
                    fa = tuple(fin[8 * k:8 * k + 4])
                    fb = tuple(fin[8 * k + 4:8 * k + 8])
                    a0 = bases[k][0] + j * 128
                    fa = _merge16(fa, plsc.load_gather(buf, [rsplats[k], a0]),
                                  a0)
                    a1 = bases[k][1] + j * 128
                    fb = _merge16(fb, plsc.load_gather(buf, [rsplats[k], a1]),
                                  a1)
                    out += list(fa) + list(fb)
                return tuple(out)
            fin = lax.fori_loop(0, _L, j_body, tuple(fin), unroll=False)

            for k in range(NR):
                a0k, a0v, a1k, a1v = fin[8 * k:8 * k + 4]
                b0k, b0v, b1k, b1v = fin[8 * k + 4:8 * k + 8]
                rb0k = lax.rev(b1k, (0,))
                rb0v = lax.rev(b1v, (0,))
                rb1k = lax.rev(b0k, (0,))
                rb1v = lax.rev(b0v, (0,))
                c0 = a0k >= rb0k
                p0k = jnp.where(c0, a0k, rb0k)
                p0v = jnp.where(c0, a0v, rb0v)
                c1 = a1k >= rb1k
                p1k = jnp.where(c1, a1k, rb1k)
                p1v = jnp.where(c1, a1v, rb1v)
                c2 = p0k >= p1k
                hik = jnp.where(c2, p0k, p1k)
                hiv = jnp.where(c2, p0v, p1v)
                lok = jnp.where(c2, p1k, p0k)
                lov = jnp.where(c2, p1v, p0v)
                f0k, f0v = plsc.sort_key_val(hik, hiv, descending=True)
                f1k, f1v = plsc.sort_key_val(lok, lov, descending=True)
                vstage[lrow0 + k, pl.ds(0, _L)] = f0k
                vstage[lrow0 + k, pl.ds(_L, _L)] = f1k
                istage[lrow0 + k, pl.ds(0, _L)] = f0v
                istage[lrow0 + k, pl.ds(_L, _L)] = f1v

        def process(buf, cmbuf, win):
            def q_body(q, _):
                r0 = q * NR
                process_quad(buf, cmbuf, r0,
                             lax.rem(win, wins_per_blk) * _W + r0)
                return 0
            lax.fori_loop(0, _W // NR, q_body, 0, unroll=False)

        pltpu.async_copy(in_slice(0), buf0, sem0)
        pltpu.async_copy(cm_slice(0), cmbuf0, csem0)

        def pair_body(t, _):
            win0 = 2 * t
            pltpu.async_copy(in_slice(win0 + 1), buf1, sem1)
            pltpu.async_copy(cm_slice(win0 + 1), cmbuf1, csem1)
            pltpu.make_async_copy(in_slice(win0), buf0, sem0).wait()
            pltpu.make_async_copy(cm_slice(win0), cmbuf0, csem0).wait()
            process(buf0, cmbuf0, win0)

            @pl.when(win0 + 2 < nwin)
            def _():
                pltpu.async_copy(in_slice(win0 + 2), buf0, sem0)
                pltpu.async_copy(cm_slice(win0 + 2), cmbuf0, csem0)

            pltpu.make_async_copy(in_slice(win0 + 1), buf1, sem1).wait()
            pltpu.make_async_copy(cm_slice(win0 + 1), cmbuf1, csem1).wait()
            process(buf1, cmbuf1, win0 + 1)

            @pl.when(lax.rem(t, pairs_per_blk) == pairs_per_blk - 1)
            def _():
                out0 = row_base + (t // pairs_per_blk) * SB
                pltpu.sync_copy(vstage, vals_hbm.at[pl.ds(out0, SB), :])
                pltpu.sync_copy(istage, idx_hbm.at[pl.ds(out0, SB), :])
            return 0
        lax.fori_loop(0, nwin // 2, pair_body, 0, unroll=False)

    return sc_topk


def kernel(source, target, num_samples, spacing):
    B, N, D = source.shape
    M = target.shape[1]
    CH = 8
    Bc = B // CH
    sc_topk = _make_sc_topk(Bc * M, N)
    vparts, iparts = [], []
    for c in range(CH):
        x, cm = _neg_dist(source, target, c * Bc, Bc)
        v, i = sc_topk(x.reshape(Bc * M, N), cm.reshape(Bc * M, N // _L))
        vparts.append(v.reshape(Bc, M, _K))
        iparts.append(i.reshape(Bc, M, _K))
    vals = jnp.concatenate(vparts, axis=0)
    idx = jnp.concatenate(iparts, axis=0)
    dep = (jnp.asarray(num_samples, dtype=idx.dtype) - _K) + jnp.asarray(
        spacing, dtype=idx.dtype)
    p_idx = idx + dep
    batch_idx = jnp.broadcast_to(
        jnp.arange(B, dtype=p_idx.dtype)[:, None, None], (B, M, _K))
    patches_idx = jnp.stack([batch_idx, p_idx], axis=-1)
    return patches_idx, vals
